# baseline (speedup 1.0000x reference)
import functools

import jax
import jax.numpy as jnp
from jax.experimental import pallas as pl

_N_BLOCK = 400


def _hosvd_body(nh_ref, ublk_ref, gblk_ref, sublk_ref, out_ref):
    bn = nh_ref.shape[0]
    r = 8
    aux = jnp.dot(nh_ref[...], ublk_ref[...], preferred_element_type=jnp.float32)
    cols = []
    a0s = []
    for g in range(3):
        a = [aux[:, 24 * c + r * g: 24 * c + r * g + r] for c in range(4)]
        c2 = (a[2][:, :, None] * a[1][:, None, :]).reshape(bn, r * r)
        x = a[3][:, :, None] * c2[:, None, :]
        cols.append(x.reshape(bn, r * r * r))
        a0s.append(jnp.broadcast_to(a[0][:, :, None], (bn, r, r)).reshape(bn, r * r))
    c3 = jnp.concatenate(cols, axis=1)
    z = jnp.dot(c3, gblk_ref[...], preferred_element_type=jnp.float32)
    p = jnp.concatenate(a0s, axis=1) * z
    out_ref[...] = jnp.dot(p, sublk_ref[...], preferred_element_type=jnp.float32)


@functools.partial(jax.jit, static_argnames=())
def kernel(neighbour_h, U0, U1, U2, U3, G_i, G_o, G_u,
           Ui_output, Uo_output, Uu_output):
    n, d, h = neighbour_h.shape
    r = G_i.shape[0]
    nh2 = neighbour_h.reshape(n, d * h)

    ublk = jax.scipy.linalg.block_diag(U0, U1, U2, U3)

    def gq(g):
        return jnp.transpose(g, (2, 3, 4, 0, 1)).reshape(r ** 3, r * r)

    gblk = jax.scipy.linalg.block_diag(gq(G_i), gq(G_o), gq(G_u))
    sublk = jax.scipy.linalg.block_diag(
        jnp.tile(Ui_output, (r, 1)),
        jnp.tile(Uo_output, (r, 1)),
        jnp.tile(Uu_output, (r, 1)))

    bn = _N_BLOCK
    grid = (n // bn,)
    out = pl.pallas_call(
        _hosvd_body,
        grid=grid,
        in_specs=[
            pl.BlockSpec((bn, d * h), lambda i: (i, 0)),
            pl.BlockSpec(ublk.shape, lambda i: (0, 0)),
            pl.BlockSpec(gblk.shape, lambda i: (0, 0)),
            pl.BlockSpec(sublk.shape, lambda i: (0, 0)),
        ],
        out_specs=pl.BlockSpec((bn, 3 * h), lambda i: (i, 0)),
        out_shape=jax.ShapeDtypeStruct((n, 3 * h), jnp.float32),
    )(nh2, ublk, gblk, sublk)
    return out


# speedup vs baseline: 4.1096x; 4.1096x over previous
import functools

import jax
import jax.numpy as jnp
from jax.experimental import pallas as pl

_N_BLOCK = 400


def _hosvd_body(nh_ref, u0e_ref, u1e_ref, u2e_ref, u3e_ref,
                gblk_ref, sublk_ref, out_ref):
    h = 256
    h0 = nh_ref[:, 0 * h:1 * h]
    h1 = nh_ref[:, 1 * h:2 * h]
    h2 = nh_ref[:, 2 * h:3 * h]
    h3 = nh_ref[:, 3 * h:4 * h]
    dot = functools.partial(jnp.dot, preferred_element_type=jnp.float32)
    a1 = dot(h1, u1e_ref[...])
    a2 = dot(h2, u2e_ref[...])
    a3 = dot(h3, u3e_ref[...])
    c3 = (a1 * a2 * a3).astype(jnp.bfloat16)
    z = dot(c3, gblk_ref[...])
    a0 = dot(h0, u0e_ref[...])
    p = (a0 * z).astype(jnp.bfloat16)
    out_ref[...] = dot(p, sublk_ref[...])


def kernel(neighbour_h, U0, U1, U2, U3, G_i, G_o, G_u,
           Ui_output, Uo_output, Uu_output):
    n, d, h = neighbour_h.shape
    r = G_i.shape[0]
    r2, r3 = r * r, r * r * r
    nh2 = neighbour_h.reshape(n, d * h).astype(jnp.bfloat16)

    def exp_cols(u, which):
        blocks = []
        for g in range(3):
            ug = u[:, r * g:r * g + r]
            if which == 'm':
                b = jnp.tile(ug, (1, r2))
            elif which == 'l':
                b = jnp.tile(jnp.repeat(ug, r, axis=1), (1, r))
            else:
                b = jnp.repeat(ug, r2, axis=1)
            blocks.append(b)
        return jnp.concatenate(blocks, axis=1)

    u1e = exp_cols(U1, 'm').astype(jnp.bfloat16)
    u2e = exp_cols(U2, 'l').astype(jnp.bfloat16)
    u3e = exp_cols(U3, 'k').astype(jnp.bfloat16)
    u0e = jnp.concatenate(
        [jnp.repeat(U0[:, r * g:r * g + r], r, axis=1) for g in range(3)],
        axis=1).astype(jnp.bfloat16)

    def gq(g):
        return jnp.transpose(g, (2, 3, 4, 0, 1)).reshape(r3, r2)

    gblk = jax.scipy.linalg.block_diag(
        gq(G_i), gq(G_o), gq(G_u)).astype(jnp.bfloat16)
    sublk = jax.scipy.linalg.block_diag(
        jnp.tile(Ui_output, (r, 1)),
        jnp.tile(Uo_output, (r, 1)),
        jnp.tile(Uu_output, (r, 1))).astype(jnp.bfloat16)

    bn = _N_BLOCK
    grid = (n // bn,)
    out = pl.pallas_call(
        _hosvd_body,
        grid=grid,
        in_specs=[
            pl.BlockSpec((bn, d * h), lambda i: (i, 0)),
            pl.BlockSpec(u0e.shape, lambda i: (0, 0)),
            pl.BlockSpec(u1e.shape, lambda i: (0, 0)),
            pl.BlockSpec(u2e.shape, lambda i: (0, 0)),
            pl.BlockSpec(u3e.shape, lambda i: (0, 0)),
            pl.BlockSpec(gblk.shape, lambda i: (0, 0)),
            pl.BlockSpec(sublk.shape, lambda i: (0, 0)),
        ],
        out_specs=pl.BlockSpec((bn, 3 * h), lambda i: (i, 0)),
        out_shape=jax.ShapeDtypeStruct((n, 3 * h), jnp.float32),
    )(nh2, u0e, u1e, u2e, u3e, gblk, sublk)
    return out


# speedup vs baseline: 4.3125x; 1.0494x over previous
import functools

import jax
import jax.numpy as jnp
from jax.experimental import pallas as pl

_N_BLOCK = 1000


def _hosvd_body(nh_ref, u0e_ref, u1e_ref, u2e_ref, u3e_ref,
                gblk_ref, sublk_ref, out_ref):
    h = 256
    h0 = nh_ref[:, 0 * h:1 * h]
    h1 = nh_ref[:, 1 * h:2 * h]
    h2 = nh_ref[:, 2 * h:3 * h]
    h3 = nh_ref[:, 3 * h:4 * h]
    dot = functools.partial(jnp.dot, preferred_element_type=jnp.float32)
    a1 = dot(h1, u1e_ref[...])
    a2 = dot(h2, u2e_ref[...])
    a3 = dot(h3, u3e_ref[...])
    c3 = (a1 * a2 * a3).astype(jnp.bfloat16)
    z = dot(c3, gblk_ref[...])
    a0 = dot(h0, u0e_ref[...])
    p = (a0 * z).astype(jnp.bfloat16)
    out_ref[...] = dot(p, sublk_ref[...])


def kernel(neighbour_h, U0, U1, U2, U3, G_i, G_o, G_u,
           Ui_output, Uo_output, Uu_output):
    n, d, h = neighbour_h.shape
    r = G_i.shape[0]
    r2, r3 = r * r, r * r * r
    nh2 = neighbour_h.reshape(n, d * h).astype(jnp.bfloat16)

    def exp_cols(u, which):
        blocks = []
        for g in range(3):
            ug = u[:, r * g:r * g + r]
            if which == 'm':
                b = jnp.tile(ug, (1, r2))
            elif which == 'l':
                b = jnp.tile(jnp.repeat(ug, r, axis=1), (1, r))
            else:
                b = jnp.repeat(ug, r2, axis=1)
            blocks.append(b)
        return jnp.concatenate(blocks, axis=1)

    u1e = exp_cols(U1, 'm').astype(jnp.bfloat16)
    u2e = exp_cols(U2, 'l').astype(jnp.bfloat16)
    u3e = exp_cols(U3, 'k').astype(jnp.bfloat16)
    u0e = jnp.concatenate(
        [jnp.repeat(U0[:, r * g:r * g + r], r, axis=1) for g in range(3)],
        axis=1).astype(jnp.bfloat16)

    def gq(g):
        return jnp.transpose(g, (2, 3, 4, 0, 1)).reshape(r3, r2)

    gblk = jax.scipy.linalg.block_diag(
        gq(G_i), gq(G_o), gq(G_u)).astype(jnp.bfloat16)
    sublk = jax.scipy.linalg.block_diag(
        jnp.tile(Ui_output, (r, 1)),
        jnp.tile(Uo_output, (r, 1)),
        jnp.tile(Uu_output, (r, 1))).astype(jnp.bfloat16)

    bn = _N_BLOCK
    grid = (n // bn,)
    out = pl.pallas_call(
        _hosvd_body,
        grid=grid,
        in_specs=[
            pl.BlockSpec((bn, d * h), lambda i: (i, 0)),
            pl.BlockSpec(u0e.shape, lambda i: (0, 0)),
            pl.BlockSpec(u1e.shape, lambda i: (0, 0)),
            pl.BlockSpec(u2e.shape, lambda i: (0, 0)),
            pl.BlockSpec(u3e.shape, lambda i: (0, 0)),
            pl.BlockSpec(gblk.shape, lambda i: (0, 0)),
            pl.BlockSpec(sublk.shape, lambda i: (0, 0)),
        ],
        out_specs=pl.BlockSpec((bn, 3 * h), lambda i: (i, 0)),
        out_shape=jax.ShapeDtypeStruct((n, 3 * h), jnp.float32),
    )(nh2, u0e, u1e, u2e, u3e, gblk, sublk)
    return out


# speedup vs baseline: 4.6666x; 1.0821x over previous
import functools

import jax
import jax.numpy as jnp
from jax.experimental import pallas as pl

_N_BLOCK = 1000


def _hosvd_body(nh_ref, u0e_ref, u1e_ref, u2e_ref, u3e_ref,
                gblk_ref, sublk_ref, out_ref):
    h = 256
    nh = nh_ref[...].astype(jnp.bfloat16)
    h0 = nh[:, 0 * h:1 * h]
    h1 = nh[:, 1 * h:2 * h]
    h2 = nh[:, 2 * h:3 * h]
    h3 = nh[:, 3 * h:4 * h]
    dot = functools.partial(jnp.dot, preferred_element_type=jnp.float32)
    a1 = dot(h1, u1e_ref[...])
    a2 = dot(h2, u2e_ref[...])
    a3 = dot(h3, u3e_ref[...])
    c3 = (a1 * a2 * a3).astype(jnp.bfloat16)
    z = dot(c3, gblk_ref[...])
    a0 = dot(h0, u0e_ref[...])
    p = (a0 * z).astype(jnp.bfloat16)
    out_ref[...] = dot(p, sublk_ref[...])


def kernel(neighbour_h, U0, U1, U2, U3, G_i, G_o, G_u,
           Ui_output, Uo_output, Uu_output):
    n, d, h = neighbour_h.shape
    r = G_i.shape[0]
    r2, r3 = r * r, r * r * r
    nh2 = neighbour_h.reshape(n, d * h)

    def exp_cols(u, which):
        blocks = []
        for g in range(3):
            ug = u[:, r * g:r * g + r]
            if which == 'm':
                b = jnp.tile(ug, (1, r2))
            elif which == 'l':
                b = jnp.tile(jnp.repeat(ug, r, axis=1), (1, r))
            else:
                b = jnp.repeat(ug, r2, axis=1)
            blocks.append(b)
        return jnp.concatenate(blocks, axis=1)

    u1e = exp_cols(U1, 'm').astype(jnp.bfloat16)
    u2e = exp_cols(U2, 'l').astype(jnp.bfloat16)
    u3e = exp_cols(U3, 'k').astype(jnp.bfloat16)
    u0e = jnp.concatenate(
        [jnp.repeat(U0[:, r * g:r * g + r], r, axis=1) for g in range(3)],
        axis=1).astype(jnp.bfloat16)

    def gq(g):
        return jnp.transpose(g, (2, 3, 4, 0, 1)).reshape(r3, r2)

    gblk = jax.scipy.linalg.block_diag(
        gq(G_i), gq(G_o), gq(G_u)).astype(jnp.bfloat16)
    sublk = jax.scipy.linalg.block_diag(
        jnp.tile(Ui_output, (r, 1)),
        jnp.tile(Uo_output, (r, 1)),
        jnp.tile(Uu_output, (r, 1))).astype(jnp.bfloat16)

    bn = _N_BLOCK
    grid = (n // bn,)
    out = pl.pallas_call(
        _hosvd_body,
        grid=grid,
        in_specs=[
            pl.BlockSpec((bn, d * h), lambda i: (i, 0)),
            pl.BlockSpec(u0e.shape, lambda i: (0, 0)),
            pl.BlockSpec(u1e.shape, lambda i: (0, 0)),
            pl.BlockSpec(u2e.shape, lambda i: (0, 0)),
            pl.BlockSpec(u3e.shape, lambda i: (0, 0)),
            pl.BlockSpec(gblk.shape, lambda i: (0, 0)),
            pl.BlockSpec(sublk.shape, lambda i: (0, 0)),
        ],
        out_specs=pl.BlockSpec((bn, 3 * h), lambda i: (i, 0)),
        out_shape=jax.ShapeDtypeStruct((n, 3 * h), jnp.float32),
    )(nh2, u0e, u1e, u2e, u3e, gblk, sublk)
    return out


# speedup vs baseline: 4.6725x; 1.0013x over previous
import functools

import jax
import jax.numpy as jnp
from jax.experimental import pallas as pl
from jax.experimental.pallas import tpu as pltpu

_N_BLOCK = 1000


def _hosvd_body(nh_ref, u0e_ref, u1e_ref, u2e_ref, u3e_ref,
                gblk_ref, sublk_ref, out_ref):
    h = 256
    nh = nh_ref[...].astype(jnp.bfloat16)
    h0 = nh[:, 0 * h:1 * h]
    h1 = nh[:, 1 * h:2 * h]
    h2 = nh[:, 2 * h:3 * h]
    h3 = nh[:, 3 * h:4 * h]
    dot = functools.partial(jnp.dot, preferred_element_type=jnp.float32)
    a1 = dot(h1, u1e_ref[...])
    a2 = dot(h2, u2e_ref[...])
    a3 = dot(h3, u3e_ref[...])
    c3 = (a1 * a2 * a3).astype(jnp.bfloat16)
    z = dot(c3, gblk_ref[...])
    a0 = dot(h0, u0e_ref[...])
    p = (a0 * z).astype(jnp.bfloat16)
    out_ref[...] = dot(p, sublk_ref[...])


def kernel(neighbour_h, U0, U1, U2, U3, G_i, G_o, G_u,
           Ui_output, Uo_output, Uu_output):
    n, d, h = neighbour_h.shape
    r = G_i.shape[0]
    r2, r3 = r * r, r * r * r
    nh2 = neighbour_h.reshape(n, d * h)

    def exp_cols(u, which):
        blocks = []
        for g in range(3):
            ug = u[:, r * g:r * g + r]
            if which == 'm':
                b = jnp.tile(ug, (1, r2))
            elif which == 'l':
                b = jnp.tile(jnp.repeat(ug, r, axis=1), (1, r))
            else:
                b = jnp.repeat(ug, r2, axis=1)
            blocks.append(b)
        return jnp.concatenate(blocks, axis=1)

    u1e = exp_cols(U1, 'm').astype(jnp.bfloat16)
    u2e = exp_cols(U2, 'l').astype(jnp.bfloat16)
    u3e = exp_cols(U3, 'k').astype(jnp.bfloat16)
    u0e = jnp.concatenate(
        [jnp.repeat(U0[:, r * g:r * g + r], r, axis=1) for g in range(3)],
        axis=1).astype(jnp.bfloat16)

    def gq(g):
        return jnp.transpose(g, (2, 3, 4, 0, 1)).reshape(r3, r2)

    gblk = jax.scipy.linalg.block_diag(
        gq(G_i), gq(G_o), gq(G_u)).astype(jnp.bfloat16)
    sublk = jax.scipy.linalg.block_diag(
        jnp.tile(Ui_output, (r, 1)),
        jnp.tile(Uo_output, (r, 1)),
        jnp.tile(Uu_output, (r, 1))).astype(jnp.bfloat16)

    bn = _N_BLOCK
    grid = (n // bn,)
    out = pl.pallas_call(
        _hosvd_body,
        grid=grid,
        in_specs=[
            pl.BlockSpec((bn, d * h), lambda i: (i, 0)),
            pl.BlockSpec(u0e.shape, lambda i: (0, 0)),
            pl.BlockSpec(u1e.shape, lambda i: (0, 0)),
            pl.BlockSpec(u2e.shape, lambda i: (0, 0)),
            pl.BlockSpec(u3e.shape, lambda i: (0, 0)),
            pl.BlockSpec(gblk.shape, lambda i: (0, 0)),
            pl.BlockSpec(sublk.shape, lambda i: (0, 0)),
        ],
        out_specs=pl.BlockSpec((bn, 3 * h), lambda i: (i, 0)),
        out_shape=jax.ShapeDtypeStruct((n, 3 * h), jnp.float32),
        compiler_params=pltpu.CompilerParams(
            dimension_semantics=("parallel",)),
    )(nh2, u0e, u1e, u2e, u3e, gblk, sublk)
    return out


# speedup vs baseline: 5.7011x; 1.2201x over previous
import functools

import jax
import jax.numpy as jnp
from jax.experimental import pallas as pl
from jax.experimental.pallas import tpu as pltpu

_N_BLOCK = 1000


def _hosvd_body(nh_ref, u0e_ref, u1e_ref, u2e_ref, u3e_ref,
                gblk_ref, sublk_ref, out_ref):
    nh = nh_ref[...].astype(jnp.bfloat16)
    h0 = nh[:, 0, :]
    h1 = nh[:, 1, :]
    h2 = nh[:, 2, :]
    h3 = nh[:, 3, :]
    dot = functools.partial(jnp.dot, preferred_element_type=jnp.float32)
    a1 = dot(h1, u1e_ref[...])
    a2 = dot(h2, u2e_ref[...])
    a3 = dot(h3, u3e_ref[...])
    c3 = (a1 * a2 * a3).astype(jnp.bfloat16)
    z = dot(c3, gblk_ref[...])
    a0 = dot(h0, u0e_ref[...])
    p = (a0 * z).astype(jnp.bfloat16)
    out_ref[...] = dot(p, sublk_ref[...])


def kernel(neighbour_h, U0, U1, U2, U3, G_i, G_o, G_u,
           Ui_output, Uo_output, Uu_output):
    n, d, h = neighbour_h.shape
    r = G_i.shape[0]
    r2, r3 = r * r, r * r * r

    def exp_cols(u, which):
        blocks = []
        for g in range(3):
            ug = u[:, r * g:r * g + r]
            if which == 'm':
                b = jnp.tile(ug, (1, r2))
            elif which == 'l':
                b = jnp.tile(jnp.repeat(ug, r, axis=1), (1, r))
            else:
                b = jnp.repeat(ug, r2, axis=1)
            blocks.append(b)
        return jnp.concatenate(blocks, axis=1)

    u1e = exp_cols(U1, 'm').astype(jnp.bfloat16)
    u2e = exp_cols(U2, 'l').astype(jnp.bfloat16)
    u3e = exp_cols(U3, 'k').astype(jnp.bfloat16)
    u0e = jnp.concatenate(
        [jnp.repeat(U0[:, r * g:r * g + r], r, axis=1) for g in range(3)],
        axis=1).astype(jnp.bfloat16)

    def gq(g):
        return jnp.transpose(g, (2, 3, 4, 0, 1)).reshape(r3, r2)

    gblk = jax.scipy.linalg.block_diag(
        gq(G_i), gq(G_o), gq(G_u)).astype(jnp.bfloat16)
    sublk = jax.scipy.linalg.block_diag(
        jnp.tile(Ui_output, (r, 1)),
        jnp.tile(Uo_output, (r, 1)),
        jnp.tile(Uu_output, (r, 1))).astype(jnp.bfloat16)

    bn = _N_BLOCK
    grid = (n // bn,)
    out = pl.pallas_call(
        _hosvd_body,
        grid=grid,
        in_specs=[
            pl.BlockSpec((bn, d, h), lambda i: (i, 0, 0)),
            pl.BlockSpec(u0e.shape, lambda i: (0, 0)),
            pl.BlockSpec(u1e.shape, lambda i: (0, 0)),
            pl.BlockSpec(u2e.shape, lambda i: (0, 0)),
            pl.BlockSpec(u3e.shape, lambda i: (0, 0)),
            pl.BlockSpec(gblk.shape, lambda i: (0, 0)),
            pl.BlockSpec(sublk.shape, lambda i: (0, 0)),
        ],
        out_specs=pl.BlockSpec((bn, 3 * h), lambda i: (i, 0)),
        out_shape=jax.ShapeDtypeStruct((n, 3 * h), jnp.float32),
        compiler_params=pltpu.CompilerParams(
            dimension_semantics=("parallel",)),
    )(neighbour_h, u0e, u1e, u2e, u3e, gblk, sublk)
    return out
